# baseline (speedup 1.0000x reference)
import jax
import jax.numpy as jnp
from jax import lax
from jax.experimental import pallas as pl
from jax.experimental.pallas import tpu as pltpu
from jax.experimental.pallas import tpu_sc as plsc

B = 16384
D = 106
NC, NS = 2, 16
NW = NC * NS
RPW = B // NW
LANES = 16
NBATCH = RPW // LANES
TAB_ROWS = 23
EMB = 32
TSTRIDE = 33


def _sc_body(key_hbm, time_hbm, mode_hbm, s0, s1, s2, s3, s4, s5, s6, s7,
             s8, s9, tab_hbm, out_hbm, kv, tv, mv, tab_v, outb, sem):
    wid = lax.axis_index("c") * NS + lax.axis_index("s")
    base = wid * RPW

    c_tab = pltpu.async_copy(tab_hbm, tab_v, sem)
    c_k = pltpu.async_copy(key_hbm.at[pl.ds(base, RPW)], kv, sem)
    c_t = pltpu.async_copy(time_hbm.at[pl.ds(base, RPW)], tv, sem)
    c_m = pltpu.async_copy(mode_hbm.at[pl.ds(base, RPW)], mv, sem)
    scal_copies = []
    for j, s in enumerate((s0, s1, s2, s3, s4, s5, s6, s7, s8, s9)):
        scal_copies.append(pltpu.async_copy(
            s.at[pl.ds(base, RPW)], outb.at[2 * EMB + j], sem))
    c_tab.wait()
    c_k.wait()
    c_t.wait()
    c_m.wait()

    def half(lo, hi, unroll=4):
        @plsc.parallel_loop(lo, hi, 1, unroll=unroll)
        def batch(i):
            b16 = i * LANES
            k = kv[pl.ds(b16, LANES)]
            t = tv[pl.ds(b16, LANES)]
            m = mv[pl.ds(b16, LANES)]
            kidx = jnp.where((k >= -1) & (k <= 11), k + 2, 0)
            tidx = jnp.where((t >= 3) & (t <= 7), t - 2, 0) + 14
            midx = jnp.where((m >= 0) & (m <= 1), m + 1, 0) + 20
            kbase = kidx * TSTRIDE
            tbase = tidx * TSTRIDE
            mbase = midx * TSTRIDE
            for c in range(EMB):
                outb[c, pl.ds(b16, LANES)] = plsc.load_gather(
                    tab_v, [kbase + c])
            for c in range(EMB):
                outb[EMB + c, pl.ds(b16, LANES)] = plsc.load_gather(
                    tab_v, [tbase + c])
            for c in range(EMB):
                outb[2 * EMB + 10 + c, pl.ds(b16, LANES)] = plsc.load_gather(
                    tab_v, [mbase + c])

    half(0, NBATCH)
    for c in scal_copies:
        c.wait()
    pltpu.sync_copy(outb, out_hbm.at[:, pl.ds(base, RPW)])


@jax.jit
def _run(key, time_signature, mode, s0, s1, s2, s3, s4, s5, s6, s7, s8, s9,
         tab):
    mesh = plsc.VectorSubcoreMesh(core_axis_name="c", subcore_axis_name="s",
                                  num_cores=NC, num_subcores=NS)
    f = pl.kernel(
        _sc_body,
        out_type=jax.ShapeDtypeStruct((D, B), jnp.float32),
        mesh=mesh,
        compiler_params=pltpu.CompilerParams(use_tc_tiling_on_sc=True,
                                             needs_layout_passes=False),
        scratch_types=(
            [pltpu.VMEM((RPW,), jnp.int32) for _ in range(3)]
            + [pltpu.VMEM((TAB_ROWS * TSTRIDE,), jnp.float32),
               pltpu.VMEM((D, RPW), jnp.float32),
               pltpu.SemaphoreType.DMA]
        ),
    )
    return f(key, time_signature, mode, s0, s1, s2, s3, s4, s5, s6, s7, s8,
             s9, tab).T


def kernel(key, time_signature, mode, danceability, energy, loudness,
           speechiness, acousticness, instrumentalness, liveness, valence,
           tempo, duration_ms, key_table, time_table, mode_table):
    tab = jnp.concatenate([key_table, time_table, mode_table])
    tab = jnp.pad(tab, ((0, 0), (0, TSTRIDE - EMB))).reshape(-1)
    return _run(key.astype(jnp.int32), time_signature.astype(jnp.int32),
                mode.astype(jnp.int32), danceability, energy, loudness,
                speechiness, acousticness, instrumentalness, liveness,
                valence, tempo, duration_ms, tab)


# speedup vs baseline: 1.1121x; 1.1121x over previous
import jax
import jax.numpy as jnp
from jax import lax
from jax.experimental import pallas as pl
from jax.experimental.pallas import tpu as pltpu
from jax.experimental.pallas import tpu_sc as plsc

B = 16384
D = 106
NC, NS = 2, 16
NW = NC * NS
RPW = B // NW
LANES = 16
NBATCH = RPW // LANES
TAB_ROWS = 23
EMB = 32
TSTRIDE = 33


def _sc_body(key_hbm, time_hbm, mode_hbm, s0, s1, s2, s3, s4, s5, s6, s7,
             s8, s9, tab_hbm, out_hbm, kv, tv, mv, tab_v, outb, sem):
    wid = lax.axis_index("c") * NS + lax.axis_index("s")
    base = wid * RPW

    c_tab = pltpu.async_copy(tab_hbm, tab_v, sem)
    c_k = pltpu.async_copy(key_hbm.at[pl.ds(base, RPW)], kv, sem)
    c_t = pltpu.async_copy(time_hbm.at[pl.ds(base, RPW)], tv, sem)
    c_m = pltpu.async_copy(mode_hbm.at[pl.ds(base, RPW)], mv, sem)
    scal_copies = []
    for j, s in enumerate((s0, s1, s2, s3, s4, s5, s6, s7, s8, s9)):
        scal_copies.append(pltpu.async_copy(
            s.at[pl.ds(base, RPW)], outb.at[2 * EMB + j], sem))
    c_tab.wait()
    c_k.wait()
    c_t.wait()
    c_m.wait()

    def half(lo, hi, unroll=1):
        @plsc.parallel_loop(lo, hi, 1, unroll=unroll)
        def batch(i):
            b16 = i * LANES
            k = kv[pl.ds(b16, LANES)]
            t = tv[pl.ds(b16, LANES)]
            m = mv[pl.ds(b16, LANES)]
            kidx = jnp.where((k >= -1) & (k <= 11), k + 2, 0)
            tidx = jnp.where((t >= 3) & (t <= 7), t - 2, 0) + 14
            midx = jnp.where((m >= 0) & (m <= 1), m + 1, 0) + 20
            kbase = kidx * TSTRIDE
            tbase = tidx * TSTRIDE
            mbase = midx * TSTRIDE
            for c in range(EMB):
                outb[c, pl.ds(b16, LANES)] = plsc.load_gather(
                    tab_v, [kbase + c])
            for c in range(EMB):
                outb[EMB + c, pl.ds(b16, LANES)] = plsc.load_gather(
                    tab_v, [tbase + c])
            for c in range(EMB):
                outb[2 * EMB + 10 + c, pl.ds(b16, LANES)] = plsc.load_gather(
                    tab_v, [mbase + c])

    half(0, NBATCH)
    for c in scal_copies:
        c.wait()
    pltpu.sync_copy(outb, out_hbm.at[:, pl.ds(base, RPW)])


@jax.jit
def _run(key, time_signature, mode, s0, s1, s2, s3, s4, s5, s6, s7, s8, s9,
         tab):
    mesh = plsc.VectorSubcoreMesh(core_axis_name="c", subcore_axis_name="s",
                                  num_cores=NC, num_subcores=NS)
    f = pl.kernel(
        _sc_body,
        out_type=jax.ShapeDtypeStruct((D, B), jnp.float32),
        mesh=mesh,
        compiler_params=pltpu.CompilerParams(use_tc_tiling_on_sc=True,
                                             needs_layout_passes=False),
        scratch_types=(
            [pltpu.VMEM((RPW,), jnp.int32) for _ in range(3)]
            + [pltpu.VMEM((TAB_ROWS * TSTRIDE,), jnp.float32),
               pltpu.VMEM((D, RPW), jnp.float32),
               pltpu.SemaphoreType.DMA]
        ),
    )
    return f(key, time_signature, mode, s0, s1, s2, s3, s4, s5, s6, s7, s8,
             s9, tab).T


def kernel(key, time_signature, mode, danceability, energy, loudness,
           speechiness, acousticness, instrumentalness, liveness, valence,
           tempo, duration_ms, key_table, time_table, mode_table):
    tab = jnp.concatenate([key_table, time_table, mode_table])
    tab = jnp.pad(tab, ((0, 0), (0, TSTRIDE - EMB))).reshape(-1)
    return _run(key.astype(jnp.int32), time_signature.astype(jnp.int32),
                mode.astype(jnp.int32), danceability, energy, loudness,
                speechiness, acousticness, instrumentalness, liveness,
                valence, tempo, duration_ms, tab)


# speedup vs baseline: 1.1412x; 1.0261x over previous
import jax
import jax.numpy as jnp
from jax import lax
from jax.experimental import pallas as pl
from jax.experimental.pallas import tpu as pltpu
from jax.experimental.pallas import tpu_sc as plsc

B = 16384
D = 106
NC, NS = 2, 16
NW = NC * NS
RPW = B // NW
LANES = 16
NBATCH = RPW // LANES
TAB_ROWS = 23
EMB = 32
TSTRIDE = 33


def _sc_body(key_hbm, time_hbm, mode_hbm, s0, s1, s2, s3, s4, s5, s6, s7,
             s8, s9, tab_hbm, out_hbm, kv, tv, mv, tab_v, outb, sem):
    wid = lax.axis_index("c") * NS + lax.axis_index("s")
    base = wid * RPW

    c_tab = pltpu.async_copy(tab_hbm, tab_v, sem)
    c_k = pltpu.async_copy(key_hbm.at[pl.ds(base, RPW)], kv, sem)
    c_t = pltpu.async_copy(time_hbm.at[pl.ds(base, RPW)], tv, sem)
    c_m = pltpu.async_copy(mode_hbm.at[pl.ds(base, RPW)], mv, sem)
    scal_copies = []
    for j, s in enumerate((s0, s1, s2, s3, s4, s5, s6, s7, s8, s9)):
        scal_copies.append(pltpu.async_copy(
            s.at[pl.ds(base, RPW)], outb.at[2 * EMB + j], sem))
    c_tab.wait()
    c_k.wait()
    c_t.wait()
    c_m.wait()

    @plsc.parallel_loop(0, NBATCH, 1, unroll=1)
    def batch_kt(i):
        b16 = i * LANES
        k = kv[pl.ds(b16, LANES)]
        t = tv[pl.ds(b16, LANES)]
        kidx = jnp.where((k >= -1) & (k <= 11), k + 2, 0)
        tidx = jnp.where((t >= 3) & (t <= 7), t - 2, 0) + 14
        kbase = kidx * TSTRIDE
        tbase = tidx * TSTRIDE
        for c in range(EMB):
            outb[c, pl.ds(b16, LANES)] = plsc.load_gather(
                tab_v, [kbase + c])
        for c in range(EMB):
            outb[EMB + c, pl.ds(b16, LANES)] = plsc.load_gather(
                tab_v, [tbase + c])

    out1 = pltpu.async_copy(outb.at[pl.ds(0, 2 * EMB)],
                            out_hbm.at[pl.ds(0, 2 * EMB), pl.ds(base, RPW)],
                            sem)

    @plsc.parallel_loop(0, NBATCH, 1, unroll=1)
    def batch_m(i):
        b16 = i * LANES
        m = mv[pl.ds(b16, LANES)]
        midx = jnp.where((m >= 0) & (m <= 1), m + 1, 0) + 20
        mbase = midx * TSTRIDE
        for c in range(EMB):
            outb[2 * EMB + 10 + c, pl.ds(b16, LANES)] = plsc.load_gather(
                tab_v, [mbase + c])

    for c in scal_copies:
        c.wait()
    out2 = pltpu.async_copy(
        outb.at[pl.ds(2 * EMB, D - 2 * EMB)],
        out_hbm.at[pl.ds(2 * EMB, D - 2 * EMB), pl.ds(base, RPW)], sem)
    out1.wait()
    out2.wait()


@jax.jit
def _run(key, time_signature, mode, s0, s1, s2, s3, s4, s5, s6, s7, s8, s9,
         tab):
    mesh = plsc.VectorSubcoreMesh(core_axis_name="c", subcore_axis_name="s",
                                  num_cores=NC, num_subcores=NS)
    f = pl.kernel(
        _sc_body,
        out_type=jax.ShapeDtypeStruct((D, B), jnp.float32),
        mesh=mesh,
        compiler_params=pltpu.CompilerParams(use_tc_tiling_on_sc=True,
                                             needs_layout_passes=False),
        scratch_types=(
            [pltpu.VMEM((RPW,), jnp.int32) for _ in range(3)]
            + [pltpu.VMEM((TAB_ROWS * TSTRIDE,), jnp.float32),
               pltpu.VMEM((D, RPW), jnp.float32),
               pltpu.SemaphoreType.DMA]
        ),
    )
    return f(key, time_signature, mode, s0, s1, s2, s3, s4, s5, s6, s7, s8,
             s9, tab).T


def kernel(key, time_signature, mode, danceability, energy, loudness,
           speechiness, acousticness, instrumentalness, liveness, valence,
           tempo, duration_ms, key_table, time_table, mode_table):
    tab = jnp.concatenate([key_table, time_table, mode_table])
    tab = jnp.pad(tab, ((0, 0), (0, TSTRIDE - EMB))).reshape(-1)
    return _run(key.astype(jnp.int32), time_signature.astype(jnp.int32),
                mode.astype(jnp.int32), danceability, energy, loudness,
                speechiness, acousticness, instrumentalness, liveness,
                valence, tempo, duration_ms, tab)


# speedup vs baseline: 1.1624x; 1.0186x over previous
import jax
import jax.numpy as jnp
from jax import lax
from jax.experimental import pallas as pl
from jax.experimental.pallas import tpu as pltpu
from jax.experimental.pallas import tpu_sc as plsc

B = 16384
D = 106
NC, NS = 2, 16
NW = NC * NS
RPW = B // NW
LANES = 16
NBATCH = RPW // LANES
TAB_ROWS = 23
EMB = 32
TSTRIDE = 33


def _sc_body(key_hbm, time_hbm, mode_hbm, s0, s1, s2, s3, s4, s5, s6, s7,
             s8, s9, tab_hbm, out_hbm, kv, tv, mv, tab_v, outb, sem):
    wid = lax.axis_index("c") * NS + lax.axis_index("s")
    base = wid * RPW

    c_tab = pltpu.async_copy(tab_hbm, tab_v, sem)
    c_k = pltpu.async_copy(key_hbm.at[pl.ds(base, RPW)], kv, sem)
    c_t = pltpu.async_copy(time_hbm.at[pl.ds(base, RPW)], tv, sem)
    c_m = pltpu.async_copy(mode_hbm.at[pl.ds(base, RPW)], mv, sem)
    scal_copies = []
    for j, s in enumerate((s0, s1, s2, s3, s4, s5, s6, s7, s8, s9)):
        scal_copies.append(pltpu.async_copy(
            s.at[pl.ds(base, RPW)], outb.at[2 * EMB + j], sem))
    c_tab.wait()
    c_k.wait()
    c_t.wait()
    c_m.wait()

    @plsc.parallel_loop(0, NBATCH, 1, unroll=1)
    def batch_k(i):
        b16 = i * LANES
        k = kv[pl.ds(b16, LANES)]
        kidx = jnp.where((k >= -1) & (k <= 11), k + 2, 0)
        kbase = kidx * TSTRIDE
        for c in range(EMB):
            outb[c, pl.ds(b16, LANES)] = plsc.load_gather(
                tab_v, [kbase + c])

    out0 = pltpu.async_copy(outb.at[pl.ds(0, EMB)],
                            out_hbm.at[pl.ds(0, EMB), pl.ds(base, RPW)],
                            sem)

    @plsc.parallel_loop(0, NBATCH, 1, unroll=1)
    def batch_t(i):
        b16 = i * LANES
        t = tv[pl.ds(b16, LANES)]
        tidx = jnp.where((t >= 3) & (t <= 7), t - 2, 0) + 14
        tbase = tidx * TSTRIDE
        for c in range(EMB):
            outb[EMB + c, pl.ds(b16, LANES)] = plsc.load_gather(
                tab_v, [tbase + c])

    out1 = pltpu.async_copy(outb.at[pl.ds(EMB, EMB)],
                            out_hbm.at[pl.ds(EMB, EMB), pl.ds(base, RPW)],
                            sem)

    @plsc.parallel_loop(0, NBATCH, 1, unroll=1)
    def batch_m(i):
        b16 = i * LANES
        m = mv[pl.ds(b16, LANES)]
        midx = jnp.where((m >= 0) & (m <= 1), m + 1, 0) + 20
        mbase = midx * TSTRIDE
        for c in range(EMB):
            outb[2 * EMB + 10 + c, pl.ds(b16, LANES)] = plsc.load_gather(
                tab_v, [mbase + c])

    for c in scal_copies:
        c.wait()
    out2 = pltpu.async_copy(
        outb.at[pl.ds(2 * EMB, D - 2 * EMB)],
        out_hbm.at[pl.ds(2 * EMB, D - 2 * EMB), pl.ds(base, RPW)], sem)
    out0.wait()
    out1.wait()
    out2.wait()


@jax.jit
def _run(key, time_signature, mode, s0, s1, s2, s3, s4, s5, s6, s7, s8, s9,
         tab):
    mesh = plsc.VectorSubcoreMesh(core_axis_name="c", subcore_axis_name="s",
                                  num_cores=NC, num_subcores=NS)
    f = pl.kernel(
        _sc_body,
        out_type=jax.ShapeDtypeStruct((D, B), jnp.float32),
        mesh=mesh,
        compiler_params=pltpu.CompilerParams(use_tc_tiling_on_sc=True,
                                             needs_layout_passes=False),
        scratch_types=(
            [pltpu.VMEM((RPW,), jnp.int32) for _ in range(3)]
            + [pltpu.VMEM((TAB_ROWS * TSTRIDE,), jnp.float32),
               pltpu.VMEM((D, RPW), jnp.float32),
               pltpu.SemaphoreType.DMA]
        ),
    )
    return f(key, time_signature, mode, s0, s1, s2, s3, s4, s5, s6, s7, s8,
             s9, tab).T


def kernel(key, time_signature, mode, danceability, energy, loudness,
           speechiness, acousticness, instrumentalness, liveness, valence,
           tempo, duration_ms, key_table, time_table, mode_table):
    tab = jnp.concatenate([key_table, time_table, mode_table])
    tab = jnp.pad(tab, ((0, 0), (0, TSTRIDE - EMB))).reshape(-1)
    return _run(key.astype(jnp.int32), time_signature.astype(jnp.int32),
                mode.astype(jnp.int32), danceability, energy, loudness,
                speechiness, acousticness, instrumentalness, liveness,
                valence, tempo, duration_ms, tab)
